# baseline (speedup 1.0000x reference)
import functools

import jax
import jax.numpy as jnp
from jax import lax
from jax.experimental import pallas as pl
from jax.experimental.pallas import tpu as pltpu
from jax.experimental.pallas import tpu_sc as plsc

D_MODEL_ = 64
SCALE_ = 8.0

_IDX_MINOR = 128
_GATHERS_PER_CHUNK = 8
_CHUNK = _IDX_MINOR * _GATHERS_PER_CHUNK


def _make_emb(n_idx_rows: int, vocab: int, d: int):
  info = plsc.get_sparse_core_info()
  nc, ns, nl = info.num_cores, info.num_subcores, info.num_lanes
  nw = nc * ns
  total = n_idx_rows * _IDX_MINOR
  per_w = total // nw
  n_chunks = per_w // _CHUNK
  assert per_w % _CHUNK == 0 and d % nl == 0

  mesh = plsc.VectorSubcoreMesh(core_axis_name="c", subcore_axis_name="s")

  @functools.partial(
      pl.kernel,
      mesh=mesh,
      compiler_params=pltpu.CompilerParams(use_tc_tiling_on_sc=False),
      out_type=jax.ShapeDtypeStruct((total, d), jnp.float32),
      scratch_types=[
          pltpu.VMEM((_GATHERS_PER_CHUNK, _IDX_MINOR), jnp.int32),
          pltpu.VMEM((_CHUNK, d), jnp.float32),
          pltpu.SemaphoreType.DMA,
      ],
  )
  def emb(idx_hbm, table_hbm, out_hbm, idx_v, rows_v, sem):
    wid = lax.axis_index("s") * nc + lax.axis_index("c")
    idx_row0 = wid * (per_w // _IDX_MINOR)
    out_row0 = wid * per_w

    def chunk_body(c, carry):
      rb = idx_row0 + c * _GATHERS_PER_CHUNK
      ob = out_row0 + c * _CHUNK
      pltpu.sync_copy(idx_hbm.at[pl.ds(rb, _GATHERS_PER_CHUNK)], idx_v)
      descs = [
          pltpu.async_copy(
              table_hbm.at[idx_v.at[j]],
              rows_v.at[pl.ds(j * _IDX_MINOR, _IDX_MINOR)],
              sem,
          )
          for j in range(_GATHERS_PER_CHUNK)
      ]
      for dsc in descs:
        dsc.wait()

      def scale_row(r, carry2):
        for j in range(d // nl):
          s = pl.ds(j * nl, nl)
          rows_v[r, s] = rows_v[r, s] * SCALE_
        return carry2

      lax.fori_loop(0, _CHUNK, scale_row, 0)
      pltpu.sync_copy(rows_v, out_hbm.at[pl.ds(ob, _CHUNK)])
      return carry

    lax.fori_loop(0, n_chunks, chunk_body, 0)

  return emb


def kernel(x, table):
  b, s = x.shape
  vocab, d = table.shape
  total = b * s
  idx2d = x.reshape(total // _IDX_MINOR, _IDX_MINOR)
  emb = _make_emb(total // _IDX_MINOR, vocab, d)
  out = emb(idx2d, table)
  return out.reshape(b, s, d)


# speedup vs baseline: 1.0737x; 1.0737x over previous
import functools

import jax
import jax.numpy as jnp
from jax import lax
from jax.experimental import pallas as pl
from jax.experimental.pallas import tpu as pltpu
from jax.experimental.pallas import tpu_sc as plsc

SCALE_ = 8.0

_IDX_MINOR = 128
_GATHERS_PER_CHUNK = 5
_CHUNK = _IDX_MINOR * _GATHERS_PER_CHUNK


def _make_emb(n_idx_rows: int, vocab: int, d: int):
  info = plsc.get_sparse_core_info()
  nc, ns, nl = info.num_cores, info.num_subcores, info.num_lanes
  nw = nc * ns
  total = n_idx_rows * _IDX_MINOR
  per_w = total // nw
  idx_rows_w = per_w // _IDX_MINOR
  n_chunks = per_w // _CHUNK
  assert per_w % _CHUNK == 0 and n_chunks % 2 == 0 and d % nl == 0

  mesh = plsc.VectorSubcoreMesh(core_axis_name="c", subcore_axis_name="s")

  @functools.partial(
      pl.kernel,
      mesh=mesh,
      compiler_params=pltpu.CompilerParams(use_tc_tiling_on_sc=False),
      out_type=jax.ShapeDtypeStruct((total, d), jnp.float32),
      scratch_types=[
          pltpu.VMEM((idx_rows_w, _IDX_MINOR), jnp.int32),
          pltpu.VMEM((_CHUNK, d), jnp.float32),
          pltpu.VMEM((_CHUNK, d), jnp.float32),
          pltpu.SemaphoreType.DMA,
          pltpu.SemaphoreType.DMA,
          pltpu.SemaphoreType.DMA,
          pltpu.SemaphoreType.DMA,
      ],
  )
  def emb(idx_hbm, table_hbm, out_hbm, idx_v, rows0, rows1,
          sem_g0, sem_g1, sem_w0, sem_w1):
    wid = lax.axis_index("s") * nc + lax.axis_index("c")
    out_row0 = wid * per_w
    bufs = ((rows0, sem_g0, sem_w0), (rows1, sem_g1, sem_w1))

    pltpu.sync_copy(idx_hbm.at[pl.ds(wid * idx_rows_w, idx_rows_w)], idx_v)

    def fire(c, rows, sem):
      for j in range(_GATHERS_PER_CHUNK):
        pltpu.async_copy(
            table_hbm.at[idx_v.at[c * _GATHERS_PER_CHUNK + j]],
            rows.at[pl.ds(j * _IDX_MINOR, _IDX_MINOR)],
            sem,
        )

    def drain(rows, sem):
      pltpu.make_async_copy(out_hbm.at[pl.ds(0, _CHUNK)], rows, sem).wait()

    fire(0, rows0, sem_g0)
    fire(1, rows1, sem_g1)

    def chunk_pair(k, carry):
      for b, (rows, sem_g, sem_w) in enumerate(bufs):
        c = 2 * k + b
        drain(rows, sem_g)

        def scale_row(r, carry2):
          for j in range(d // nl):
            s = pl.ds(j * nl, nl)
            rows[r, s] = rows[r, s] * SCALE_
          return carry2

        lax.fori_loop(0, _CHUNK, scale_row, 0)
        pltpu.async_copy(rows, out_hbm.at[pl.ds(out_row0 + c * _CHUNK, _CHUNK)],
                         sem_w)

      @pl.when(k < n_chunks // 2 - 1)
      def _prefetch():
        for b, (rows, sem_g, sem_w) in enumerate(bufs):
          drain(rows, sem_w)
          fire(2 * k + b + 2, rows, sem_g)

      return carry

    lax.fori_loop(0, n_chunks // 2, chunk_pair, 0)
    drain(rows0, sem_w0)
    drain(rows1, sem_w1)

  return emb


def kernel(x, table):
  b, s = x.shape
  vocab, d = table.shape
  total = b * s
  idx2d = x.reshape(total // _IDX_MINOR, _IDX_MINOR)
  emb = _make_emb(total // _IDX_MINOR, vocab, d)
  out = emb(idx2d, table)
  return out.reshape(b, s, d)
